# Initial kernel scaffold; baseline (speedup 1.0000x reference)
import jax
import jax.numpy as jnp
from jax.experimental import pallas as pl


def kernel(x, edge_indices, edge_weights, batch, pre_params, graph_params, post_params):
    raise NotImplementedError("write your pallas kernel here")


# speedup vs baseline: 3.2260x; 3.2260x over previous
import functools

import jax
import jax.numpy as jnp
from jax import lax
from jax.experimental import pallas as pl
from jax.experimental.pallas import tpu as pltpu
from jax.experimental.pallas import tpu_sc as plsc

N = 10000
E = 320000
DF = 128
D = 64
G = 64

NC = 2
NS = 16
NW = NC * NS
K = 128
NCHUNK = 80
E_PAD = NW * NCHUNK * K
RPT = N // NS


def _relu(v):
    return jnp.maximum(v, 0.0)


def _sc_aggregate(x, src3, dst3, ew3):
    mesh = plsc.VectorSubcoreMesh(core_axis_name="c", subcore_axis_name="s")

    @functools.partial(
        pl.kernel,
        mesh=mesh,
        compiler_params=pltpu.CompilerParams(use_tc_tiling_on_sc=False),
        out_type=jax.ShapeDtypeStruct((NC, NS, RPT, D), jnp.float32),
        scratch_types=[
            pltpu.VMEM((NCHUNK, K), jnp.int32),
            pltpu.VMEM((NCHUNK, K), jnp.int32),
            pltpu.VMEM((NCHUNK, K), jnp.float32),
            pltpu.VMEM((K, D), jnp.float32),
            pltpu.VMEM((RPT, D), jnp.float32),
            pltpu.VMEM_SHARED((N, D), jnp.float32),
            pltpu.SemaphoreType.DMA,
        ],
    )
    def agg(x_hbm, src_hbm, dst_hbm, ew_hbm, out_hbm,
            src_v, dst_v, ew_v, rows_v, stage_v, acc_sh, sem):
        c = lax.axis_index("c")
        s = lax.axis_index("s")
        wid = s * NC + c

        def zrow(i, carry):
            for j in range(D // 16):
                stage_v[i, pl.ds(j * 16, 16)] = jnp.zeros((16,), jnp.float32)
            return carry
        lax.fori_loop(0, RPT, zrow, 0)
        pltpu.sync_copy(stage_v, acc_sh.at[pl.ds(s * RPT, RPT)])
        plsc.subcore_barrier()

        pltpu.sync_copy(src_hbm.at[wid], src_v)
        pltpu.sync_copy(dst_hbm.at[wid], dst_v)
        pltpu.sync_copy(ew_hbm.at[wid], ew_v)

        def chunk(ci, carry):
            pltpu.async_copy(x_hbm.at[src_v.at[ci]], rows_v, sem).wait()

            def group(g2, c2):
                base_e = g2 * 16
                w16 = ew_v[ci, pl.ds(base_e, 16)]
                for e in range(16):
                    wv = lax.broadcast_in_dim(w16[e:e + 1], (16,), (0,))
                    r = base_e + e
                    for j in range(D // 16):
                        rows_v[r, pl.ds(j * 16, 16)] = (
                            rows_v[r, pl.ds(j * 16, 16)] * wv)
                return c2
            lax.fori_loop(0, K // 16, group, 0)

            pltpu.sync_copy(rows_v, acc_sh.at[dst_v.at[ci]], add=True)
            return carry
        lax.fori_loop(0, NCHUNK, chunk, 0)

        plsc.subcore_barrier()
        pltpu.sync_copy(acc_sh.at[pl.ds(s * RPT, RPT)], stage_v)
        pltpu.sync_copy(stage_v, out_hbm.at[c, s])

    return agg(x, src3, dst3, ew3).reshape(NC, N, D)


def _pre_body(x_ref, *refs):
    out_ref = refs[-1]
    h = x_ref[...]
    for i in range(3):
        wa, ba, wb, bb = refs[4 * i:4 * i + 4]
        h = _relu(jnp.dot(h, wa[...], preferred_element_type=jnp.float32)
                  + ba[...])
        h = _relu(jnp.dot(h, wb[...], preferred_element_type=jnp.float32)
                  + bb[...])
    out_ref[...] = h


def _pre_mlp(x, pre_params):
    flat = []
    for (w1, b1), (w2, b2) in pre_params:
        flat += [w1, b1.reshape(1, -1), w2, b2.reshape(1, -1)]
    return pl.pallas_call(
        _pre_body,
        out_shape=jax.ShapeDtypeStruct((N, D), jnp.float32),
    )(x, *flat)


def _bn(z, g, b):
    m = jnp.mean(z, axis=0, keepdims=True)
    v = jnp.mean((z - m) ** 2, axis=0, keepdims=True)
    return g * (z - m) / jnp.sqrt(v + 1e-5) + b


def _gin_body(parts_ref, w1, b1, g1, be1, w2, b2, g2, be2, batch_ref,
              xout_ref, pool_ref):
    aggr = parts_ref[0] + parts_ref[1]
    h = _relu(_bn(jnp.dot(aggr, w1[...], preferred_element_type=jnp.float32)
                  + b1[...], g1[...], be1[...]))
    h = _relu(_bn(jnp.dot(h, w2[...], preferred_element_type=jnp.float32)
                  + b2[...], g2[...], be2[...]))
    xout_ref[...] = h
    onehot_t = (lax.broadcasted_iota(jnp.int32, (G, N), 0)
                == batch_ref[...]).astype(jnp.float32)
    pool_ref[...] = jnp.dot(onehot_t, h, preferred_element_type=jnp.float32)


def _gin_layer(parts, lp, batch_row):
    (w1, b1), (g1, be1), (w2, b2), (g2, be2) = lp
    return pl.pallas_call(
        _gin_body,
        out_shape=(
            jax.ShapeDtypeStruct((N, D), jnp.float32),
            jax.ShapeDtypeStruct((G, D), jnp.float32),
        ),
    )(parts, w1, b1.reshape(1, -1), g1.reshape(1, -1), be1.reshape(1, -1),
      w2, b2.reshape(1, -1), g2.reshape(1, -1), be2.reshape(1, -1), batch_row)


def _post_body(*refs):
    pooled = refs[:6]
    (wa, ba, wb, bb, wc, bc, wd, bd, we, be, wf, bf) = refs[6:18]
    out_ref = refs[18]
    hc = jnp.concatenate([p[...] for p in pooled], axis=1)
    hc = _relu(jnp.dot(hc, wa[...], preferred_element_type=jnp.float32) + ba[...])
    hc = _relu(jnp.dot(hc, wb[...], preferred_element_type=jnp.float32) + bb[...])
    hc = _relu(jnp.dot(hc, wc[...], preferred_element_type=jnp.float32) + bc[...])
    hc = _relu(jnp.dot(hc, wd[...], preferred_element_type=jnp.float32) + bd[...])
    hc = _relu(jnp.dot(hc, we[...], preferred_element_type=jnp.float32) + be[...])
    out_ref[...] = (jnp.dot(hc, wf[...], preferred_element_type=jnp.float32)
                    + bf[...])


def _post_mlp(pooled, post_params):
    flat = []
    for (w1, b1), (w2, b2) in post_params:
        flat += [w1, b1.reshape(1, -1), w2, b2.reshape(1, -1)]
    return pl.pallas_call(
        _post_body,
        out_shape=jax.ShapeDtypeStruct((G, 1), jnp.float32),
    )(*pooled, *flat)


def kernel(x, edge_indices, edge_weights, batch, pre_params, graph_params,
           post_params):
    src = edge_indices[0]
    dst = edge_indices[1]
    pad = E_PAD - E
    src3 = jnp.concatenate([src, jnp.zeros((pad,), jnp.int32)]).reshape(
        NW, NCHUNK, K)
    dst3 = jnp.concatenate([dst, jnp.zeros((pad,), jnp.int32)]).reshape(
        NW, NCHUNK, K)
    ew3 = jnp.concatenate(
        [edge_weights, jnp.zeros((pad,), jnp.float32)]).reshape(NW, NCHUNK, K)
    batch_row = batch.reshape(1, N)

    h = _pre_mlp(x, pre_params)
    pooled = []
    for lp in graph_params:
        parts = _sc_aggregate(h, src3, dst3, ew3)
        h, pool_l = _gin_layer(parts, lp, batch_row)
        pooled.append(pool_l)
    return _post_mlp(pooled, post_params)


# speedup vs baseline: 4.3338x; 1.3434x over previous
import functools

import jax
import jax.numpy as jnp
import numpy as np
from jax import lax
from jax.experimental import pallas as pl
from jax.experimental.pallas import tpu as pltpu
from jax.experimental.pallas import tpu_sc as plsc

N = 10000
E = 320000
DF = 128
D = 64
G = 64

NC = 2
NS = 16
NW = NC * NS
K = 128
NCHUNK = 80
SLAB = NCHUNK * K
NBLK = 79
NBPT = 5
TRASH0 = 10016

_SIZES = np.array(([10240] * 4 + [9920] * 12) * 2, np.int64)
_STARTS = np.concatenate([[0], np.cumsum(_SIZES)[:-1]])


def _relu(v):
    return jnp.maximum(v, 0.0)


def _sc_aggregate(x, src3, idx3, ew3, keep3, tail2):
    mesh = plsc.VectorSubcoreMesh(core_axis_name="c", subcore_axis_name="s")

    @functools.partial(
        pl.kernel,
        mesh=mesh,
        compiler_params=pltpu.CompilerParams(use_tc_tiling_on_sc=False),
        out_type=jax.ShapeDtypeStruct((NC, NBLK, K, D), jnp.float32),
        scratch_types=[
            pltpu.VMEM((NCHUNK, K), jnp.int32),
            pltpu.VMEM((NCHUNK, K), jnp.int32),
            pltpu.VMEM((NCHUNK, K), jnp.float32),
            pltpu.VMEM((NCHUNK, K), jnp.float32),
            pltpu.VMEM((1, K), jnp.int32),
            pltpu.VMEM((D,), jnp.float32),
            [pltpu.VMEM((K, D), jnp.float32) for _ in range(4)],
            [pltpu.VMEM((K, D), jnp.float32) for _ in range(2)],
            pltpu.VMEM_SHARED((NBLK * K, D), jnp.float32),
            [pltpu.SemaphoreType.DMA for _ in range(4)],
            [pltpu.SemaphoreType.DMA for _ in range(2)],
        ],
    )
    def agg(x_hbm, src_hbm, idx_hbm, ew_hbm, keep_hbm, tail_hbm, out_hbm,
            src_v, idx_v, ew_v, keep_v, tidx_v, cur_v, rows, stag, acc_sh, gsem, ssem):
        c = lax.axis_index("c")
        s = lax.axis_index("s")
        wid = s * NC + c

        for b in range(2):
            def zrow(i, carry, b=b):
                for j in range(D // 16):
                    rows[b][i, pl.ds(j * 16, 16)] = jnp.zeros((16,), jnp.float32)
                return carry
            lax.fori_loop(0, K, zrow, 0)
        for k in range(NBPT):
            blk = s * NBPT + k

            @pl.when(blk < NBLK)
            def _(blk=blk, k=k):
                pltpu.sync_copy(rows[k % 2], acc_sh.at[pl.ds(blk * K, K)])
        plsc.subcore_barrier()

        pltpu.sync_copy(src_hbm.at[wid], src_v)
        pltpu.sync_copy(idx_hbm.at[wid], idx_v)
        pltpu.sync_copy(ew_hbm.at[wid], ew_v)
        pltpu.sync_copy(keep_hbm.at[wid], keep_v)

        def walk_chunk(rows_b, stag_b, ci):
            def group(g2, carry):
                base_e = g2 * 16
                w16 = ew_v[ci, pl.ds(base_e, 16)]
                k16 = keep_v[ci, pl.ds(base_e, 16)]
                cs = [cur_v[pl.ds(j * 16, 16)] for j in range(D // 16)]
                for e in range(16):
                    r = base_e + e
                    wv = lax.broadcast_in_dim(w16[e:e + 1], (16,), (0,))
                    kv = lax.broadcast_in_dim(k16[e:e + 1], (16,), (0,))
                    ncs = []
                    for j in range(D // 16):
                        stag_b[r, pl.ds(j * 16, 16)] = cs[j]
                        u = rows_b[r, pl.ds(j * 16, 16)] * wv
                        ncs.append(cs[j] * kv + u)
                    cs = ncs
                for j in range(D // 16):
                    cur_v[pl.ds(j * 16, 16)] = cs[j]
                return carry
            lax.fori_loop(0, K // 16, group, 0)

        pltpu.async_copy(x_hbm.at[src_v.at[0]], rows[0], gsem[0])
        pltpu.async_copy(x_hbm.at[src_v.at[1]], rows[1], gsem[1])

        for j in range(D // 16):
            cur_v[pl.ds(j * 16, 16)] = jnp.zeros((16,), jnp.float32)

        def outer(g, carry):
            for b in range(4):
                ci = g * 4 + b
                pltpu.make_async_copy(
                    x_hbm.at[src_v.at[ci]], rows[b], gsem[b]).wait()

                @pl.when(ci + 2 < NCHUNK)
                def _():
                    pltpu.async_copy(x_hbm.at[src_v.at[ci + 2]],
                                     rows[(b + 2) % 4], gsem[(b + 2) % 4])
                sb = b % 2

                @pl.when(ci >= 2)
                def _():
                    pltpu.make_async_copy(
                        stag[sb], acc_sh.at[idx_v.at[0]], ssem[sb]).wait()
                walk_chunk(rows[b], stag[sb], ci)
                pltpu.async_copy(
                    stag[sb], acc_sh.at[idx_v.at[ci]], ssem[sb], add=True)
            return carry
        lax.fori_loop(0, NCHUNK // 4, outer, 0)
        pltpu.make_async_copy(stag[0], acc_sh.at[idx_v.at[0]], ssem[0]).wait()
        pltpu.make_async_copy(stag[1], acc_sh.at[idx_v.at[0]], ssem[1]).wait()

        pltpu.sync_copy(tail_hbm.at[wid], tidx_v)
        for j in range(D // 16):
            stag[0][0, pl.ds(j * 16, 16)] = cur_v[pl.ds(j * 16, 16)]
        pltpu.sync_copy(stag[0], acc_sh.at[tidx_v.at[0]], add=True)

        plsc.subcore_barrier()
        for k in range(NBPT):
            blk = s * NBPT + k

            @pl.when(blk < NBLK)
            def _(blk=blk, k=k):
                pltpu.sync_copy(acc_sh.at[pl.ds(blk * K, K)], rows[k % 4])
                pltpu.sync_copy(rows[k % 4], out_hbm.at[c, blk])

    return agg(x, src3, idx3, ew3, keep3, tail2).reshape(NC, NBLK * K, D)[:, :N]


def _pre_body(x_ref, *refs):
    out_ref = refs[-1]
    h = x_ref[...]
    for i in range(3):
        wa, ba, wb, bb = refs[4 * i:4 * i + 4]
        h = _relu(jnp.dot(h, wa[...], preferred_element_type=jnp.float32)
                  + ba[...])
        h = _relu(jnp.dot(h, wb[...], preferred_element_type=jnp.float32)
                  + bb[...])
    out_ref[...] = h


def _pre_mlp(x, pre_params):
    flat = []
    for (w1, b1), (w2, b2) in pre_params:
        flat += [w1, b1.reshape(1, -1), w2, b2.reshape(1, -1)]
    return pl.pallas_call(
        _pre_body,
        out_shape=jax.ShapeDtypeStruct((N, D), jnp.float32),
    )(x, *flat)


_INVN = np.float32(1.0) / np.float32(N)


def _mean16(ref):
    def body(k, accs):
        base = k * 128
        return tuple(accs[j] + ref[pl.ds(base + 8 * j, 8), :]
                     for j in range(16))
    accs = lax.fori_loop(0, N // 128, body,
                         tuple(jnp.zeros((8, D), jnp.float32)
                               for _ in range(16)))
    accs = list(accs)
    accs[0] = accs[0] + ref[pl.ds(N - 16, 8), :]
    accs[1] = accs[1] + ref[pl.ds(N - 8, 8), :]
    acc = accs[0]
    for j in range(1, 16):
        acc = acc + accs[j]
    t = acc[:4] + acc[4:]
    t = t[:2] + t[2:]
    s = t[:1] + t[1:]
    return s * _INVN


def _gin_body(parts_ref, w1, b1, g1, be1, w2, b2, g2, be2, batch_ref,
              xout_ref, pool_ref, zscr, dscr):
    aggr = parts_ref[0] + parts_ref[1]
    zscr[...] = jnp.dot(aggr, w1[...],
                        preferred_element_type=jnp.float32) + b1[...]
    m = _mean16(zscr)
    d = zscr[...] - m
    dscr[...] = d * d
    v = _mean16(dscr)
    h = _relu(g1[...] * d / jnp.sqrt(v + 1e-5) + be1[...])
    zscr[...] = jnp.dot(h, w2[...],
                        preferred_element_type=jnp.float32) + b2[...]
    m = _mean16(zscr)
    d = zscr[...] - m
    dscr[...] = d * d
    v = _mean16(dscr)
    h = _relu(g2[...] * d / jnp.sqrt(v + 1e-5) + be2[...])
    xout_ref[...] = h
    onehot_t = (lax.broadcasted_iota(jnp.int32, (G, N), 0)
                == batch_ref[...]).astype(jnp.float32)
    pool_ref[...] = jnp.dot(onehot_t, h, preferred_element_type=jnp.float32,
                            precision=jax.lax.Precision.HIGHEST)


def _gin_layer(parts, lp, batch_row):
    (w1, b1), (g1, be1), (w2, b2), (g2, be2) = lp
    return pl.pallas_call(
        _gin_body,
        out_shape=(
            jax.ShapeDtypeStruct((N, D), jnp.float32),
            jax.ShapeDtypeStruct((G, D), jnp.float32),
        ),
        scratch_shapes=[pltpu.VMEM((N, D), jnp.float32),
                        pltpu.VMEM((N, D), jnp.float32)],
    )(parts, w1, b1.reshape(1, -1), g1.reshape(1, -1), be1.reshape(1, -1),
      w2, b2.reshape(1, -1), g2.reshape(1, -1), be2.reshape(1, -1), batch_row)


def _post_body(*refs):
    pooled = refs[:6]
    (wa, ba, wb, bb, wc, bc, wd, bd, we, be, wf, bf) = refs[6:18]
    out_ref = refs[18]
    hc = jnp.concatenate([p[...] for p in pooled], axis=1)
    hc = _relu(jnp.dot(hc, wa[...], preferred_element_type=jnp.float32) + ba[...])
    hc = _relu(jnp.dot(hc, wb[...], preferred_element_type=jnp.float32) + bb[...])
    hc = _relu(jnp.dot(hc, wc[...], preferred_element_type=jnp.float32) + bc[...])
    hc = _relu(jnp.dot(hc, wd[...], preferred_element_type=jnp.float32) + bd[...])
    hc = _relu(jnp.dot(hc, we[...], preferred_element_type=jnp.float32) + be[...])
    out_ref[...] = (jnp.dot(hc, wf[...], preferred_element_type=jnp.float32)
                    + bf[...])


def _post_mlp(pooled, post_params):
    flat = []
    for (w1, b1), (w2, b2) in post_params:
        flat += [w1, b1.reshape(1, -1), w2, b2.reshape(1, -1)]
    return pl.pallas_call(
        _post_body,
        out_shape=jax.ShapeDtypeStruct((G, 1), jnp.float32),
    )(*pooled, *flat)


def kernel(x, edge_indices, edge_weights, batch, pre_params, graph_params,
           post_params):
    src = edge_indices[0]
    dst = edge_indices[1]

    perm = jnp.argsort(dst, stable=True)
    src_s = src[perm]
    dst_s = dst[perm]
    ew_s = edge_weights[perm]

    ii = jnp.arange(SLAB)[None, :]
    bw = jnp.asarray(_STARTS)[:, None]
    szw = jnp.asarray(_SIZES)[:, None]
    p = bw + ii
    pc = jnp.clip(p, 0, E - 1)
    real = ii < szw
    src3 = jnp.where(real, src_s[pc], 0).astype(jnp.int32)
    ew3 = jnp.where(real, ew_s[pc], 0.0).astype(jnp.float32)
    dst3 = jnp.where(real, dst_s[pc], -1)
    prev_raw = jnp.where(ii <= szw, dst_s[jnp.clip(p - 1, 0, E - 1)], -1)
    prev3 = jnp.where(ii == 0, dst3[:, :1], prev_raw)
    bit = dst3 != prev3
    trash = TRASH0 + (ii % 64)
    idx3 = jnp.where(bit, prev3, trash).astype(jnp.int32)
    keep3 = jnp.where(bit, 0.0, 1.0).astype(jnp.float32)
    tail_node = jnp.where(szw[:, 0] == SLAB,
                          dst_s[jnp.clip(bw[:, 0] + szw[:, 0] - 1, 0, E - 1)],
                          TRASH0 + 1)
    tail2 = jnp.full((NW, 1, K), TRASH0 + 2, jnp.int32).at[:, 0, 0].set(
        tail_node.astype(jnp.int32))

    src3 = src3.reshape(NW, NCHUNK, K)
    idx3 = idx3.reshape(NW, NCHUNK, K)
    ew3 = ew3.reshape(NW, NCHUNK, K)
    keep3 = keep3.reshape(NW, NCHUNK, K)
    batch_row = batch.reshape(1, N)

    h = _pre_mlp(x, pre_params)
    pooled = []
    for lp in graph_params:
        parts = _sc_aggregate(h, src3, idx3, ew3, keep3, tail2)
        h, pool_l = _gin_layer(parts, lp, batch_row)
        pooled.append(pool_l)
    return _post_mlp(pooled, post_params)
